# baseline (speedup 1.0000x reference)
import functools

import jax
import jax.numpy as jnp
from jax import lax
from jax.experimental import pallas as pl
from jax.experimental.pallas import tpu as pltpu
from jax.experimental.pallas import tpu_sc as plsc

V = 8192
NTOK = 8192
LANES = 128
SLABS = V // LANES
ROWS_PER_BLK = 512
NBLK = V // ROWS_PER_BLK

NC = 2
NS = 16
L = 16
NW = NC * NS
TPW = NTOK // NW
CH = 128
NCH = TPW // CH
SUB = TPW // L
SUBPC = CH // L


def _lse_block(x_ref, o_ref):
    x = x_ref[...]
    m = jnp.max(x, axis=(1, 2))
    s = jnp.sum(jnp.exp(x - m[:, None, None]), axis=(1, 2))
    o_ref[...] = (m + jnp.log(s))[None, None, :]


def _compute_lse(table3):
    out = pl.pallas_call(
        _lse_block,
        grid=(NBLK,),
        in_specs=[pl.BlockSpec((ROWS_PER_BLK, SLABS, LANES), lambda i: (i, 0, 0))],
        out_specs=pl.BlockSpec((1, 1, ROWS_PER_BLK), lambda i: (i, 0, 0)),
        out_shape=jax.ShapeDtypeStruct((NBLK, 1, ROWS_PER_BLK), jnp.float32),
    )(table3)
    return out.reshape(V)


@functools.cache
def _sc_gather_kernel():
    mesh = plsc.VectorSubcoreMesh(
        core_axis_name="c", subcore_axis_name="s", num_cores=NC, num_subcores=NS
    )

    @functools.partial(
        pl.kernel,
        out_type=jax.ShapeDtypeStruct((NW, L), jnp.float32),
        mesh=mesh,
        scratch_types=[
            pltpu.VMEM((TPW,), jnp.int32),
            pltpu.VMEM((TPW,), jnp.int32),
            pltpu.VMEM((NCH, CH), jnp.int32),
            pltpu.VMEM((NCH, CH, LANES), jnp.float32),
            pltpu.VMEM((V,), jnp.float32),
            pltpu.VMEM((L,), jnp.float32),
            pltpu.SemaphoreType.DMA,
        ],
        compiler_params=pltpu.CompilerParams(
            use_tc_tiling_on_sc=False, needs_layout_passes=False
        ),
    )
    def _sc_gather(prev_hbm, tgt_hbm, lse_hbm, tslab_hbm, out_hbm,
                   prev_v, tgt_v, blk_v, row_v, lse_v, acc_v, sem):
        _sc_gather_body(prev_hbm, tgt_hbm, lse_hbm, tslab_hbm, out_hbm,
                        prev_v, tgt_v, blk_v, row_v, lse_v, acc_v, sem)

    return _sc_gather


def _sc_gather_body(prev_hbm, tgt_hbm, lse_hbm, tslab_hbm, out_hbm,
                    prev_v, tgt_v, blk_v, row_v, lse_v, acc_v, sem):
    wid = lax.axis_index("s") * NC + lax.axis_index("c")
    base = wid * TPW
    pltpu.sync_copy(prev_hbm.at[pl.ds(base, TPW)], prev_v)
    pltpu.sync_copy(tgt_hbm.at[pl.ds(base, TPW)], tgt_v)
    pltpu.sync_copy(lse_hbm, lse_v)

    for j in range(SUB):
        p = prev_v[pl.ds(j * L, L)]
        t = tgt_v[pl.ds(j * L, L)]
        blk = p * SLABS + lax.shift_right_logical(t, 7)
        blk_v[j // SUBPC, pl.ds((j % SUBPC) * L, L)] = blk

    copies = [
        pltpu.async_copy(tslab_hbm.at[blk_v.at[c]], row_v.at[c], sem)
        for c in range(NCH)
    ]
    for cp in copies:
        cp.wait()

    acc = jnp.zeros((L,), jnp.float32)
    iota = lax.iota(jnp.int32, L)
    for j in range(SUB):
        p = prev_v[pl.ds(j * L, L)]
        t = tgt_v[pl.ds(j * L, L)]
        lse_g = plsc.load_gather(lse_v, [p])
        rows = iota + (j % SUBPC) * L
        offs = jnp.bitwise_and(t, LANES - 1)
        vals = plsc.load_gather(row_v.at[j // SUBPC], [rows, offs])
        acc = acc + (lse_g - vals)
    acc_v[...] = acc
    pltpu.sync_copy(acc_v, out_hbm.at[wid])


def kernel(input_ids, target_ids, transition_logits):
    prev = input_ids.reshape(-1).astype(jnp.int32)
    tgt = target_ids.reshape(-1).astype(jnp.int32)
    table = transition_logits.astype(jnp.float32)
    table3 = table.reshape(V, SLABS, LANES)
    tslab = table.reshape(V * SLABS, LANES)
    lse = _compute_lse(table3)
    partials = _sc_gather_kernel()(prev, tgt, lse, tslab)
    return jnp.sum(partials) / NTOK


# speedup vs baseline: 2.3917x; 2.3917x over previous
import functools

import jax
import jax.numpy as jnp
from jax import lax
from jax.experimental import pallas as pl
from jax.experimental.pallas import tpu as pltpu
from jax.experimental.pallas import tpu_sc as plsc

V = 8192
NTOK = 8192
LANES = 128
SLABS = V // LANES
ROWS_PER_BLK = 256
NBLK = V // ROWS_PER_BLK

NC = 2
NS = 16
L = 16
NW = NC * NS
TPW = NTOK // NW
CH = 128
NCH = TPW // CH
SUB = TPW // L
SUBPC = CH // L


def _lse_block(x_ref, o_ref, t_ref):
    x = x_ref[...]
    m = jnp.max(x, axis=1)
    s = jnp.sum(jnp.exp(x - m[:, None]), axis=1)
    o_ref[...] = (m + jnp.log(s))[None, None, :]
    t_ref[...] = x.reshape(ROWS_PER_BLK, SLABS, LANES)


def _compute_lse(table):
    lse, tslab = pl.pallas_call(
        _lse_block,
        grid=(NBLK,),
        in_specs=[pl.BlockSpec((ROWS_PER_BLK, V), lambda i: (i, 0))],
        out_specs=[
            pl.BlockSpec((1, 1, ROWS_PER_BLK), lambda i: (i, 0, 0)),
            pl.BlockSpec((ROWS_PER_BLK, SLABS, LANES), lambda i: (i, 0, 0)),
        ],
        out_shape=[
            jax.ShapeDtypeStruct((NBLK, 1, ROWS_PER_BLK), jnp.float32),
            jax.ShapeDtypeStruct((V, SLABS, LANES), jnp.float32),
        ],
    )(table)
    return lse.reshape(V), tslab.reshape(V * SLABS, LANES)


@functools.cache
def _sc_gather_kernel():
    mesh = plsc.VectorSubcoreMesh(
        core_axis_name="c", subcore_axis_name="s", num_cores=NC, num_subcores=NS
    )

    @functools.partial(
        pl.kernel,
        out_type=jax.ShapeDtypeStruct((NW, L), jnp.float32),
        mesh=mesh,
        scratch_types=[
            pltpu.VMEM((TPW,), jnp.int32),
            pltpu.VMEM((TPW,), jnp.int32),
            pltpu.VMEM((NCH, CH), jnp.int32),
            pltpu.VMEM((NCH, CH, LANES), jnp.float32),
            pltpu.VMEM((V,), jnp.float32),
            pltpu.VMEM((L,), jnp.float32),
            pltpu.SemaphoreType.DMA,
        ],
        compiler_params=pltpu.CompilerParams(
            use_tc_tiling_on_sc=False, needs_layout_passes=False
        ),
    )
    def _sc_gather(prev_hbm, tgt_hbm, lse_hbm, tslab_hbm, out_hbm,
                   prev_v, tgt_v, blk_v, row_v, lse_v, acc_v, sem):
        _sc_gather_body(prev_hbm, tgt_hbm, lse_hbm, tslab_hbm, out_hbm,
                        prev_v, tgt_v, blk_v, row_v, lse_v, acc_v, sem)

    return _sc_gather


def _sc_gather_body(prev_hbm, tgt_hbm, lse_hbm, tslab_hbm, out_hbm,
                    prev_v, tgt_v, blk_v, row_v, lse_v, acc_v, sem):
    wid = lax.axis_index("s") * NC + lax.axis_index("c")
    base = wid * TPW
    pltpu.sync_copy(prev_hbm.at[pl.ds(base, TPW)], prev_v)
    pltpu.sync_copy(tgt_hbm.at[pl.ds(base, TPW)], tgt_v)
    pltpu.sync_copy(lse_hbm, lse_v)

    for j in range(SUB):
        p = prev_v[pl.ds(j * L, L)]
        t = tgt_v[pl.ds(j * L, L)]
        blk = p * SLABS + lax.shift_right_logical(t, 7)
        blk_v[j // SUBPC, pl.ds((j % SUBPC) * L, L)] = blk

    copies = [
        pltpu.async_copy(tslab_hbm.at[blk_v.at[c]], row_v.at[c], sem)
        for c in range(NCH)
    ]
    for cp in copies:
        cp.wait()

    acc = jnp.zeros((L,), jnp.float32)
    iota = lax.iota(jnp.int32, L)
    for j in range(SUB):
        p = prev_v[pl.ds(j * L, L)]
        t = tgt_v[pl.ds(j * L, L)]
        lse_g = plsc.load_gather(lse_v, [p])
        rows = iota + (j % SUBPC) * L
        offs = jnp.bitwise_and(t, LANES - 1)
        vals = plsc.load_gather(row_v.at[j // SUBPC], [rows, offs])
        acc = acc + (lse_g - vals)
    acc_v[...] = acc
    pltpu.sync_copy(acc_v, out_hbm.at[wid])


def kernel(input_ids, target_ids, transition_logits):
    prev = input_ids.reshape(-1).astype(jnp.int32)
    tgt = target_ids.reshape(-1).astype(jnp.int32)
    table = transition_logits.astype(jnp.float32)
    lse, tslab = _compute_lse(table)
    partials = _sc_gather_kernel()(prev, tgt, lse, tslab)
    return jnp.sum(partials) / NTOK


# speedup vs baseline: 2.8257x; 1.1815x over previous
import functools

import jax
import jax.numpy as jnp
from jax import lax
from jax.experimental import pallas as pl
from jax.experimental.pallas import tpu as pltpu
from jax.experimental.pallas import tpu_sc as plsc

V = 8192
NTOK = 8192
LANES = 128
PSLABS = V // (2 * LANES)
ROWS_PER_BLK = 256
NBLK = V // ROWS_PER_BLK

NC = 2
NS = 16
L = 16
NW = NC * NS
TPW = NTOK // NW
CH = 128
NCH = TPW // CH
SUB = TPW // L
SUBPC = CH // L


def _lse_block(x_ref, o_ref, t_ref):
    x = x_ref[...]
    m = jnp.max(x, axis=1)
    s = jnp.sum(jnp.exp(x - m[:, None]), axis=1)
    o_ref[...] = (m + jnp.log(s))[None, None, :]
    u = jax.lax.bitcast_convert_type(x, jnp.int32)
    lo = u[:, :V // 2] + 0x8000
    hi = u[:, V // 2:] + 0x8000
    pk = lax.shift_right_logical(lo, 16) | (hi & jnp.int32(-65536))
    t_ref[...] = pk.reshape(pk.shape[0], PSLABS, LANES)


def _compute_lse(table):
    lse, tslab = pl.pallas_call(
        _lse_block,
        grid=(NBLK,),
        in_specs=[pl.BlockSpec((ROWS_PER_BLK, V), lambda i: (i, 0))],
        out_specs=[
            pl.BlockSpec((1, 1, ROWS_PER_BLK), lambda i: (i, 0, 0)),
            pl.BlockSpec((ROWS_PER_BLK, PSLABS, LANES), lambda i: (i, 0, 0)),
        ],
        out_shape=[
            jax.ShapeDtypeStruct((NBLK, 1, ROWS_PER_BLK), jnp.float32),
            jax.ShapeDtypeStruct((V, PSLABS, LANES), jnp.int32),
        ],
    )(table)
    return lse.reshape(V), tslab.reshape(V * PSLABS, LANES)


@functools.cache
def _sc_gather_kernel():
    mesh = plsc.VectorSubcoreMesh(
        core_axis_name="c", subcore_axis_name="s", num_cores=NC, num_subcores=NS
    )

    @functools.partial(
        pl.kernel,
        out_type=jax.ShapeDtypeStruct((NW, L), jnp.float32),
        mesh=mesh,
        scratch_types=[
            pltpu.VMEM((TPW,), jnp.int32),
            pltpu.VMEM((TPW,), jnp.int32),
            pltpu.VMEM((NCH, CH), jnp.int32),
            pltpu.VMEM((NCH, CH, LANES), jnp.int32),
            pltpu.VMEM((V,), jnp.float32),
            pltpu.VMEM((L,), jnp.float32),
            pltpu.SemaphoreType.DMA,
        ],
        compiler_params=pltpu.CompilerParams(
            use_tc_tiling_on_sc=False, needs_layout_passes=False
        ),
    )
    def _sc_gather(prev_hbm, tgt_hbm, lse_hbm, tslab_hbm, out_hbm,
                   prev_v, tgt_v, blk_v, row_v, lse_v, acc_v, sem):
        _sc_gather_body(prev_hbm, tgt_hbm, lse_hbm, tslab_hbm, out_hbm,
                        prev_v, tgt_v, blk_v, row_v, lse_v, acc_v, sem)

    return _sc_gather


def _sc_gather_body(prev_hbm, tgt_hbm, lse_hbm, tslab_hbm, out_hbm,
                    prev_v, tgt_v, blk_v, row_v, lse_v, acc_v, sem):
    wid = lax.axis_index("s") * NC + lax.axis_index("c")
    base = wid * TPW
    pltpu.sync_copy(prev_hbm.at[pl.ds(base, TPW)], prev_v)
    pltpu.sync_copy(tgt_hbm.at[pl.ds(base, TPW)], tgt_v)
    pltpu.sync_copy(lse_hbm, lse_v)

    for j in range(SUB):
        p = prev_v[pl.ds(j * L, L)]
        t = tgt_v[pl.ds(j * L, L)]
        blk = p * PSLABS + (lax.shift_right_logical(t, 7) & (PSLABS - 1))
        blk_v[j // SUBPC, pl.ds((j % SUBPC) * L, L)] = blk

    copies = [
        pltpu.async_copy(tslab_hbm.at[blk_v.at[c]], row_v.at[c], sem)
        for c in range(NCH)
    ]
    for cp in copies:
        cp.wait()

    acc = jnp.zeros((L,), jnp.float32)
    iota = lax.iota(jnp.int32, L)
    for j in range(SUB):
        p = prev_v[pl.ds(j * L, L)]
        t = tgt_v[pl.ds(j * L, L)]
        lse_g = plsc.load_gather(lse_v, [p])
        rows = iota + (j % SUBPC) * L
        offs = jnp.bitwise_and(t, LANES - 1)
        word = plsc.load_gather(row_v.at[j // SUBPC], [rows, offs])
        hi_sel = lax.shift_right_logical(t, 12)
        bits = jnp.where(
            hi_sel == 1,
            word & jnp.int32(-65536),
            lax.shift_left(word, 16),
        )
        vals = plsc.bitcast(bits, jnp.float32)
        acc = acc + (lse_g - vals)
    acc_v[...] = acc
    pltpu.sync_copy(acc_v, out_hbm.at[wid])


def kernel(input_ids, target_ids, transition_logits):
    prev = input_ids.reshape(-1).astype(jnp.int32)
    tgt = target_ids.reshape(-1).astype(jnp.int32)
    table = transition_logits.astype(jnp.float32)
    lse, tslab = _compute_lse(table)
    partials = _sc_gather_kernel()(prev, tgt, lse, tslab)
    return jnp.sum(partials) / NTOK


# speedup vs baseline: 3.5103x; 1.2423x over previous
import functools

import jax
import jax.numpy as jnp
from jax import lax
from jax.experimental import pallas as pl
from jax.experimental.pallas import tpu as pltpu
from jax.experimental.pallas import tpu_sc as plsc

V = 8192
NTOK = 8192
LANES = 128
ROWS_PER_BLK = 256
NBLK = V // ROWS_PER_BLK

NC = 2
NS = 16
L = 16
NW = NC * NS
TPW = NTOK // NW
SUB = TPW // L


def _lse_block(x_ref, o_ref):
    x = x_ref[...]
    m = jnp.max(x, axis=1)
    s = jnp.sum(jnp.exp(x - m[:, None]), axis=1)
    o_ref[...] = (m + jnp.log(s))[None, None, :]


def _compute_lse(table):
    lse = pl.pallas_call(
        _lse_block,
        grid=(NBLK,),
        in_specs=[pl.BlockSpec((ROWS_PER_BLK, V), lambda i: (i, 0))],
        out_specs=pl.BlockSpec((1, 1, ROWS_PER_BLK), lambda i: (i, 0, 0)),
        out_shape=jax.ShapeDtypeStruct((NBLK, 1, ROWS_PER_BLK), jnp.float32),
    )(table)
    return lse.reshape(V)


@functools.cache
def _val_kernel():
    mesh = plsc.VectorSubcoreMesh(
        core_axis_name="c", subcore_axis_name="s", num_cores=NC, num_subcores=NS
    )

    @functools.partial(
        pl.kernel,
        out_type=jax.ShapeDtypeStruct((NW, LANES), jnp.float32),
        mesh=mesh,
        scratch_types=[
            pltpu.VMEM((TPW,), jnp.int32),
            pltpu.VMEM((TPW,), jnp.int32),
            pltpu.VMEM((TPW, LANES), jnp.float32),
            pltpu.VMEM((LANES,), jnp.float32),
            pltpu.SemaphoreType.DMA,
        ],
        compiler_params=pltpu.CompilerParams(
            use_tc_tiling_on_sc=True, needs_layout_passes=False
        ),
    )
    def _val(tbl_hbm, prev_hbm, tgt_hbm, out_hbm, prev_v, tgt_v, slab_v,
             st_v, sem):
        wid = lax.axis_index("s") * NC + lax.axis_index("c")
        base = wid * TPW
        pltpu.sync_copy(prev_hbm.at[pl.ds(base, TPW)], prev_v)
        pltpu.sync_copy(tgt_hbm.at[pl.ds(base, TPW)], tgt_v)
        iota = lax.iota(jnp.int32, L)
        for g in range(SUB):
            p = prev_v[pl.ds(g * L, L)]
            t = tgt_v[pl.ds(g * L, L)]
            off = p * V + (t & ~jnp.int32(LANES - 1))
            copies = []
            for lane in range(L):
                o = jnp.max(jnp.where(iota == lane, off, 0))
                r = lax.shift_right_logical(o, 13)
                c = pl.multiple_of(jnp.bitwise_and(o, V - 1), LANES)
                copies.append(pltpu.async_copy(
                    tbl_hbm.at[pl.ds(r, 1), pl.ds(c, LANES)],
                    slab_v.at[pl.ds(g * L + lane, 1)], sem))
            for cp in copies:
                cp.wait()
        acc = jnp.zeros((L,), jnp.float32)
        for g in range(SUB):
            t = tgt_v[pl.ds(g * L, L)]
            rows = iota + g * L
            offs = jnp.bitwise_and(t, LANES - 1)
            vals = plsc.load_gather(slab_v, [rows, offs])
            acc = acc + vals
        zero = jnp.zeros((L,), jnp.float32)
        for i in range(LANES // L):
            st_v[pl.ds(i * L, L)] = zero
        st_v[pl.ds(0, L)] = acc
        pltpu.sync_copy(st_v, out_hbm.at[wid])

    return _val


@functools.cache
def _lse_sum_kernel():
    mesh = plsc.VectorSubcoreMesh(
        core_axis_name="c", subcore_axis_name="s", num_cores=NC, num_subcores=NS
    )

    @functools.partial(
        pl.kernel,
        out_type=jax.ShapeDtypeStruct((NW, L), jnp.float32),
        mesh=mesh,
        scratch_types=[
            pltpu.VMEM((TPW,), jnp.int32),
            pltpu.VMEM((V,), jnp.float32),
            pltpu.VMEM((L,), jnp.float32),
            pltpu.SemaphoreType.DMA,
        ],
        compiler_params=pltpu.CompilerParams(
            use_tc_tiling_on_sc=False, needs_layout_passes=False
        ),
    )
    def _lse_sum(prev_hbm, lse_hbm, out_hbm, prev_v, lse_v, acc_v, sem):
        wid = lax.axis_index("s") * NC + lax.axis_index("c")
        base = wid * TPW
        pltpu.sync_copy(prev_hbm.at[pl.ds(base, TPW)], prev_v)
        pltpu.sync_copy(lse_hbm, lse_v)
        acc = jnp.zeros((L,), jnp.float32)
        for j in range(SUB):
            p = prev_v[pl.ds(j * L, L)]
            acc = acc + plsc.load_gather(lse_v, [p])
        acc_v[...] = acc
        pltpu.sync_copy(acc_v, out_hbm.at[wid])

    return _lse_sum


def kernel(input_ids, target_ids, transition_logits):
    prev = input_ids.reshape(-1).astype(jnp.int32)
    tgt = target_ids.reshape(-1).astype(jnp.int32)
    table = transition_logits.astype(jnp.float32)
    lse = _compute_lse(table)
    val_partials = _val_kernel()(table, prev, tgt)
    lse_partials = _lse_sum_kernel()(prev, lse)
    return (jnp.sum(lse_partials) - jnp.sum(val_partials)) / NTOK


# speedup vs baseline: 3.8845x; 1.1066x over previous
import functools

import jax
import jax.numpy as jnp
from jax import lax
from jax.experimental import pallas as pl
from jax.experimental.pallas import tpu as pltpu
from jax.experimental.pallas import tpu_sc as plsc

V = 8192
NTOK = 8192
LANES = 128
ROWS_PER_BLK = 512
NBLK = V // ROWS_PER_BLK

NC = 2
NS = 16
L = 16
NW = NC * NS
TPW = NTOK // NW
SUB = TPW // L


def _lse_block(x_ref, o_ref):
    x = x_ref[...]
    m = jnp.max(x, axis=1)
    s = jnp.sum(jnp.exp(x - m[:, None]), axis=1)
    o_ref[...] = (m + jnp.log(s))[None, None, :]


def _compute_lse(table):
    lse = pl.pallas_call(
        _lse_block,
        grid=(NBLK,),
        in_specs=[pl.BlockSpec((ROWS_PER_BLK, V), lambda i: (i, 0))],
        out_specs=pl.BlockSpec((1, 1, ROWS_PER_BLK), lambda i: (i, 0, 0)),
        out_shape=jax.ShapeDtypeStruct((NBLK, 1, ROWS_PER_BLK), jnp.float32),
    )(table)
    return lse.reshape(V)


@functools.cache
def _val_kernel():
    mesh = plsc.VectorSubcoreMesh(
        core_axis_name="c", subcore_axis_name="s", num_cores=NC, num_subcores=NS
    )

    @functools.partial(
        pl.kernel,
        out_type=jax.ShapeDtypeStruct((NW, LANES), jnp.float32),
        mesh=mesh,
        scratch_types=[
            pltpu.VMEM((TPW,), jnp.int32),
            pltpu.VMEM((TPW,), jnp.int32),
            pltpu.VMEM((TPW, LANES), jnp.float32),
            pltpu.VMEM((LANES,), jnp.float32),
            pltpu.SemaphoreType.DMA,
        ],
        compiler_params=pltpu.CompilerParams(
            use_tc_tiling_on_sc=True, needs_layout_passes=False
        ),
    )
    def _val(tbl_hbm, prev_hbm, tgt_hbm, out_hbm, prev_v, tgt_v, slab_v,
             st_v, sem):
        wid = lax.axis_index("s") * NC + lax.axis_index("c")
        base = wid * TPW
        pltpu.sync_copy(prev_hbm.at[pl.ds(base, TPW)], prev_v)
        pltpu.sync_copy(tgt_hbm.at[pl.ds(base, TPW)], tgt_v)
        iota = lax.iota(jnp.int32, L)
        copies = []
        for g in range(SUB):
            p = prev_v[pl.ds(g * L, L)]
            t = tgt_v[pl.ds(g * L, L)]
            off = p * V + (t & ~jnp.int32(LANES - 1))
            for lane in range(L):
                o = jnp.max(jnp.where(iota == lane, off, 0))
                r = lax.shift_right_logical(o, 13)
                c = pl.multiple_of(jnp.bitwise_and(o, V - 1), LANES)
                copies.append(pltpu.async_copy(
                    tbl_hbm.at[pl.ds(r, 1), pl.ds(c, LANES)],
                    slab_v.at[pl.ds(g * L + lane, 1)], sem))
        for cp in copies:
            cp.wait()
        acc = jnp.zeros((L,), jnp.float32)
        for g in range(SUB):
            t = tgt_v[pl.ds(g * L, L)]
            rows = iota + g * L
            offs = jnp.bitwise_and(t, LANES - 1)
            vals = plsc.load_gather(slab_v, [rows, offs])
            acc = acc + vals
        zero = jnp.zeros((L,), jnp.float32)
        for i in range(LANES // L):
            st_v[pl.ds(i * L, L)] = zero
        st_v[pl.ds(0, L)] = acc
        pltpu.sync_copy(st_v, out_hbm.at[wid])

    return _val


@functools.cache
def _lse_sum_kernel():
    mesh = plsc.VectorSubcoreMesh(
        core_axis_name="c", subcore_axis_name="s", num_cores=NC, num_subcores=NS
    )

    @functools.partial(
        pl.kernel,
        out_type=jax.ShapeDtypeStruct((NW, L), jnp.float32),
        mesh=mesh,
        scratch_types=[
            pltpu.VMEM((TPW,), jnp.int32),
            pltpu.VMEM((V,), jnp.float32),
            pltpu.VMEM((L,), jnp.float32),
            pltpu.SemaphoreType.DMA,
        ],
        compiler_params=pltpu.CompilerParams(
            use_tc_tiling_on_sc=False, needs_layout_passes=False
        ),
    )
    def _lse_sum(prev_hbm, lse_hbm, out_hbm, prev_v, lse_v, acc_v, sem):
        wid = lax.axis_index("s") * NC + lax.axis_index("c")
        base = wid * TPW
        pltpu.sync_copy(prev_hbm.at[pl.ds(base, TPW)], prev_v)
        pltpu.sync_copy(lse_hbm, lse_v)
        acc = jnp.zeros((L,), jnp.float32)
        for j in range(SUB):
            p = prev_v[pl.ds(j * L, L)]
            acc = acc + plsc.load_gather(lse_v, [p])
        acc_v[...] = acc
        pltpu.sync_copy(acc_v, out_hbm.at[wid])

    return _lse_sum


def kernel(input_ids, target_ids, transition_logits):
    prev = input_ids.reshape(-1).astype(jnp.int32)
    tgt = target_ids.reshape(-1).astype(jnp.int32)
    table = transition_logits.astype(jnp.float32)
    lse = _compute_lse(table)
    val_partials = _val_kernel()(table, prev, tgt)
    lse_partials = _lse_sum_kernel()(prev, lse)
    return (jnp.sum(lse_partials) - jnp.sum(val_partials)) / NTOK
